# baseline (speedup 1.0000x reference)
import jax
import jax.numpy as jnp
from jax import lax
from jax.experimental import pallas as pl
from jax.experimental.pallas import tpu as pltpu
from jax.experimental.pallas import tpu_sc as plsc

NUM_ROWS = 1000000
EMB_D = 32
BATCH = 16384
SEQ = 50
NUM_TOKENS = BATCH * SEQ
CHUNK = 128
NUM_CHUNKS = NUM_TOKENS // CHUNK
NC, NS = 2, 16
NW = NC * NS
CH_PER_W = NUM_CHUNKS // NW
CHB_PER_S = BATCH // CHUNK

PSTRIDE = 1000064

KA = 800
NKA = NUM_ROWS // KA
NIA = (NKA + NW - 1) // NW

DEPTH = 8
LAG = 2
SLAB = 4
TB = 2
GROUPS = CH_PER_W // DEPTH
SLABS_PER_W = CH_PER_W // SLAB


def _iota16():
    return lax.iota(jnp.int32, 16)


def _transpose_chunk(src, dst, off):
    def body(tid, n):
        for u in range(8):
            t = tid * 8 + u
            col = jnp.full((16,), off, jnp.int32) + t
            v0 = src[t, pl.ds(0, 16)]
            plsc.store_scatter(dst, [_iota16(), col], v0)
            v1 = src[t, pl.ds(16, 16)]
            plsc.store_scatter(dst, [_iota16() + 16, col], v1)
        return n

    lax.fori_loop(0, CHUNK // 8, body, 0)


def _convert_body(table1d_hbm, rows_hbm, *bufs):
    inb = bufs[0:2]
    outb = bufs[2:4]
    isem = bufs[4:6]
    osem = bufs[6:8]
    wid = lax.axis_index("s") * NC + lax.axis_index("c")
    splats = [jnp.full((16,), d, jnp.int32) for d in range(EMB_D)]

    def chunk_of(i):
        return wid + i * NW

    def valid(i):
        return chunk_of(i) < NKA

    def load_start(i, w):
        base = chunk_of(i) * KA
        for d in range(EMB_D):
            pltpu.async_copy(table1d_hbm.at[pl.ds(d * PSTRIDE + base, KA)],
                             inb[w].at[pl.ds(d * KA, KA)], isem[w])

    def load_wait(w):
        pltpu.make_async_copy(table1d_hbm.at[pl.ds(0, EMB_D * KA)], inb[w],
                              isem[w]).wait()

    def store_start(i, w):
        pltpu.async_copy(outb[w], rows_hbm.at[pl.ds(chunk_of(i) * KA, KA)],
                         osem[w])

    def store_wait(w):
        pltpu.make_async_copy(outb[w], rows_hbm.at[pl.ds(0, KA)],
                              osem[w]).wait()

    @pl.when(valid(0))
    def _():
        load_start(0, 0)

    def step(g, _):
        for w in range(2):
            i = g * 2 + w

            @pl.when(valid(i))
            def _():
                load_wait(w)

            @pl.when(valid(i + 1))
            def _():
                load_start(i + 1, (w + 1) % 2)

            @pl.when(valid(i) & (i >= 2))
            def _():
                store_wait(w)

            @pl.when(valid(i))
            def _():
                def tr(cg, x):
                    rows = _iota16() + cg * 16
                    for d in range(EMB_D):
                        v = inb[w][pl.ds(d * KA + cg * 16, 16)]
                        plsc.store_scatter(outb[w], [rows, splats[d]], v)
                    return x
                lax.fori_loop(0, KA // 16, tr, 0)
                store_start(i, w)
        return 0

    lax.fori_loop(0, NIA // 2, step, 0)
    for k in (NIA - 2, NIA - 1):
        @pl.when(valid(k))
        def _():
            store_wait(k % 2)


def _gather_body(idx_hbm, rows_hbm, out_hbm, idx_v, *rest):
    rows = rest[:DEPTH]
    gsem = rest[DEPTH:2 * DEPTH]
    tbuf = rest[2 * DEPTH:2 * DEPTH + TB]
    ssem = rest[2 * DEPTH + TB:2 * DEPTH + 2 * TB]
    wid = lax.axis_index("s") * NC + lax.axis_index("c")
    ch_base = wid * CH_PER_W
    pltpu.sync_copy(idx_hbm.at[pl.ds(ch_base, CH_PER_W)], idx_v)

    def gather_start(chunk, slot):
        pltpu.async_copy(rows_hbm.at[idx_v.at[chunk]], rows[slot],
                         gsem[slot])

    def gather_wait(slot):
        pltpu.make_async_copy(rows_hbm.at[pl.ds(0, CHUNK)], rows[slot],
                              gsem[slot]).wait()

    def store_start(slab, slot):
        c = ch_base + slab * SLAB
        s = c // CHB_PER_S
        b0 = (c % CHB_PER_S) * CHUNK
        pltpu.async_copy(tbuf[slot],
                         out_hbm.at[s, :, pl.ds(b0, SLAB * CHUNK)],
                         ssem[slot])

    def store_wait(slot):
        pltpu.make_async_copy(tbuf[slot],
                              out_hbm.at[0, :, pl.ds(0, SLAB * CHUNK)],
                              ssem[slot]).wait()

    for m in range(DEPTH - LAG):
        gather_start(m, m)

    def group(g, _):
        for b in range(DEPTH):
            j = g * DEPTH + b
            sl = (b - LAG) % DEPTH
            if b >= LAG:
                @pl.when(g < GROUPS - 1)
                def _():
                    gather_start(j + DEPTH - LAG, sl)
            else:
                gather_start(j + DEPTH - LAG, sl)
            gather_wait(b)
            sb = b // SLAB
            slab = g * (DEPTH // SLAB) + sb
            if b % SLAB == 0:
                @pl.when(g >= 1)
                def _():
                    store_wait(sb)
            _transpose_chunk(rows[b], tbuf[sb], (b % SLAB) * CHUNK)
            if b % SLAB == SLAB - 1:
                store_start(slab, sb)
        return 0

    lax.fori_loop(0, GROUPS, group, 0)

    for i in range(TB):
        store_wait(i)


@jax.jit
def _embed(idx2d, rows_table):
    mesh = plsc.VectorSubcoreMesh(core_axis_name="c", subcore_axis_name="s")
    gather = pl.kernel(
        _gather_body,
        out_type=jax.ShapeDtypeStruct((SEQ, EMB_D, BATCH), jnp.float32),
        mesh=mesh,
        scratch_types=(
            [pltpu.VMEM((CH_PER_W, CHUNK), jnp.int32)]
            + [pltpu.VMEM((CHUNK, EMB_D), jnp.float32) for _ in range(DEPTH)]
            + [pltpu.SemaphoreType.DMA for _ in range(DEPTH)]
            + [pltpu.VMEM((EMB_D, SLAB * CHUNK), jnp.float32)
               for _ in range(TB)]
            + [pltpu.SemaphoreType.DMA for _ in range(TB)]
        ),
        compiler_params=pltpu.CompilerParams(use_tc_tiling_on_sc=False, needs_layout_passes=False),
    )
    return gather(idx2d, rows_table)


def kernel(token_ids, embedding_matrix):
    idx2d = token_ids.T.reshape(NUM_CHUNKS, CHUNK).astype(jnp.int32)
    out = _embed(idx2d, embedding_matrix)
    return out.transpose(2, 0, 1)


# speedup vs baseline: 1.0041x; 1.0041x over previous
import jax
import jax.numpy as jnp
from jax import lax
from jax.experimental import pallas as pl
from jax.experimental.pallas import tpu as pltpu
from jax.experimental.pallas import tpu_sc as plsc

NUM_ROWS = 1000000
EMB_D = 32
BATCH = 16384
SEQ = 50
NUM_TOKENS = BATCH * SEQ
CHUNK = 128
NUM_CHUNKS = NUM_TOKENS // CHUNK
NC, NS = 2, 16
NW = NC * NS
CH_PER_W = NUM_CHUNKS // NW
CHB_PER_S = BATCH // CHUNK

DEPTH = 8
LAG = 2
SLAB = 4
TB = 2
GROUPS = CH_PER_W // DEPTH
SLABS_PER_W = CH_PER_W // SLAB


def _iota16():
    return lax.iota(jnp.int32, 16)


def _transpose_chunk(src, dst, off):
    def body(tid, n):
        for u in range(8):
            t = tid * 8 + u
            col = jnp.full((16,), off, jnp.int32) + t
            v0 = src[t, pl.ds(0, 16)]
            plsc.store_scatter(dst, [_iota16(), col], v0)
            v1 = src[t, pl.ds(16, 16)]
            plsc.store_scatter(dst, [_iota16() + 16, col], v1)
        return n

    lax.fori_loop(0, CHUNK // 8, body, 0)


def _gather_body(idx_hbm, rows_hbm, out_hbm, idx_v, *rest):
    rows = rest[:DEPTH]
    gsem = rest[DEPTH:2 * DEPTH]
    tbuf = rest[2 * DEPTH:2 * DEPTH + TB]
    ssem = rest[2 * DEPTH + TB:2 * DEPTH + 2 * TB]
    wid = lax.axis_index("s") * NC + lax.axis_index("c")
    ch_base = wid * CH_PER_W
    pltpu.sync_copy(idx_hbm.at[pl.ds(ch_base, CH_PER_W)], idx_v)

    def gather_start(chunk, slot):
        pltpu.async_copy(rows_hbm.at[idx_v.at[chunk]], rows[slot],
                         gsem[slot])

    def gather_wait(slot):
        pltpu.make_async_copy(rows_hbm.at[pl.ds(0, CHUNK)], rows[slot],
                              gsem[slot]).wait()

    def store_start(slab, slot):
        c = ch_base + slab * SLAB
        s = c // CHB_PER_S
        b0 = (c % CHB_PER_S) * CHUNK
        pltpu.async_copy(tbuf[slot],
                         out_hbm.at[s, :, pl.ds(b0, SLAB * CHUNK)],
                         ssem[slot])

    def store_wait(slot):
        pltpu.make_async_copy(tbuf[slot],
                              out_hbm.at[0, :, pl.ds(0, SLAB * CHUNK)],
                              ssem[slot]).wait()

    for m in range(DEPTH - LAG):
        gather_start(m, m)

    def group(g, _):
        for b in range(DEPTH):
            j = g * DEPTH + b
            sl = (b - LAG) % DEPTH
            if b >= LAG:
                @pl.when(g < GROUPS - 1)
                def _():
                    gather_start(j + DEPTH - LAG, sl)
            else:
                gather_start(j + DEPTH - LAG, sl)
            gather_wait(b)
            sb = b // SLAB
            slab = g * (DEPTH // SLAB) + sb
            if b % SLAB == 0:
                @pl.when(g >= 1)
                def _():
                    store_wait(sb)
            _transpose_chunk(rows[b], tbuf[sb], (b % SLAB) * CHUNK)
            if b % SLAB == SLAB - 1:
                store_start(slab, sb)
        return 0

    lax.fori_loop(0, GROUPS, group, 0)

    for i in range(TB):
        store_wait(i)


@jax.jit
def _embed(idx2d, rows_table):
    mesh = plsc.VectorSubcoreMesh(core_axis_name="c", subcore_axis_name="s")
    gather = pl.kernel(
        _gather_body,
        out_type=jax.ShapeDtypeStruct((SEQ, EMB_D, BATCH), jnp.float32),
        mesh=mesh,
        scratch_types=(
            [pltpu.VMEM((CH_PER_W, CHUNK), jnp.int32)]
            + [pltpu.VMEM((CHUNK, EMB_D), jnp.float32) for _ in range(DEPTH)]
            + [pltpu.SemaphoreType.DMA for _ in range(DEPTH)]
            + [pltpu.VMEM((EMB_D, SLAB * CHUNK), jnp.float32)
               for _ in range(TB)]
            + [pltpu.SemaphoreType.DMA for _ in range(TB)]
        ),
        compiler_params=pltpu.CompilerParams(use_tc_tiling_on_sc=False, needs_layout_passes=False),
    )
    return gather(idx2d, rows_table)


def kernel(token_ids, embedding_matrix):
    idx2d = token_ids.T.reshape(NUM_CHUNKS, CHUNK).astype(jnp.int32)
    out = _embed(idx2d, embedding_matrix)
    return out.transpose(2, 0, 1)
